# baseline (speedup 1.0000x reference)
import dataclasses
import functools

import jax
import jax.numpy as jnp
from jax import lax
from jax.experimental import pallas as pl
from jax.experimental.pallas import tpu as pltpu
from jax.experimental.pallas import tpu_sc as plsc

DIM = 128
EPS = 1e-07
B = 16384
NNEG = 5
ND = NNEG + 1
NC, NS, L = 2, 16, 16
NW = NC * NS
PER_W = B // NW
CHUNK = 32
NBUF = 4
NCHUNK = PER_W // CHUNK
NSL = DIM // L


def _sc_body(pos_u_hbm, pos_v_hbm, neg_hbm, u_w_hbm, v_w_hbm, out_hbm,
             idx_u, idx_v, idx_n5, out_full,
             rows_u0, rows_v0, rows_n0,
             rows_u1, rows_v1, rows_n1,
             rows_u2, rows_v2, rows_n2,
             rows_u3, rows_v3, rows_n3,
             sem_g0, sem_g1, sem_g2, sem_g3, sem_i):
    wid = lax.axis_index("s") * NC + lax.axis_index("c")
    base = wid * PER_W
    ncopy = pltpu.make_async_copy(neg_hbm.at[:, pl.ds(base, PER_W)], idx_n5,
                                  sem_i)
    ncopy.start()
    pltpu.sync_copy(pos_u_hbm.at[pl.ds(base, PER_W)], idx_u)
    pltpu.sync_copy(pos_v_hbm.at[pl.ds(base, PER_W)], idx_v)

    bufs = ((rows_u0, rows_v0, rows_n0, sem_g0),
            (rows_u1, rows_v1, rows_n1, sem_g1),
            (rows_u2, rows_v2, rows_n2, sem_g2),
            (rows_u3, rows_v3, rows_n3, sem_g3))

    def fire_uv(c, b):
        ru, rv, _, sg = bufs[b]
        off = c * CHUNK
        pltpu.async_copy(u_w_hbm.at[idx_u.at[pl.ds(off, CHUNK)]], ru, sg)
        pltpu.async_copy(v_w_hbm.at[idx_v.at[pl.ds(off, CHUNK)]], rv, sg)

    def fire_n(c, b):
        _, _, rn, sg = bufs[b]
        off = c * CHUNK
        for k in range(NNEG):
            pltpu.async_copy(v_w_hbm.at[idx_n5.at[k, pl.ds(off, CHUNK)]],
                             rn.at[k], sg)

    def fire(c, b):
        fire_uv(c, b)
        fire_n(c, b)

    def wait_gathers(b):
        ru, rv, rn, sg = bufs[b]
        pltpu.make_async_copy(u_w_hbm.at[idx_u.at[pl.ds(0, CHUNK)]], ru, sg).wait()
        pltpu.make_async_copy(v_w_hbm.at[idx_v.at[pl.ds(0, CHUNK)]], rv, sg).wait()
        for k in range(NNEG):
            pltpu.make_async_copy(v_w_hbm.at[idx_n5.at[k, pl.ds(0, CHUNK)]],
                                  rn.at[k], sg).wait()

    def compute(c, b):
        ru, rv, rn, _ = bufs[b]
        coff = c * CHUNK

        lane = lax.iota(jnp.int32, L)

        @plsc.parallel_loop(0, CHUNK // L, unroll=2)
        def _grp(g):
            def body(j, carry):
                i = g * L + j
                sel = lane == j
                us = [ru[i, pl.ds(s * L, L)] for s in range(NSL)]
                acc = us[0] * rv[i, pl.ds(0, L)]
                for s in range(1, NSL):
                    acc += us[s] * rv[i, pl.ds(s * L, L)]
                outs = [jnp.where(sel, jnp.sum(acc), carry[0])]
                for k in range(NNEG):
                    acc = us[0] * rn[k, i, pl.ds(0, L)]
                    for s in range(1, NSL):
                        acc += us[s] * rn[k, i, pl.ds(s * L, L)]
                    outs.append(jnp.where(sel, jnp.sum(acc), carry[1 + k]))
                return tuple(outs)

            zero = jnp.zeros((L,), jnp.float32)
            dots = lax.fori_loop(0, L, body, (zero,) * ND)
            for d in range(ND):
                out_full[d, pl.ds(coff + g * L, L)] = dots[d]

    for b in range(NBUF):
        fire_uv(b, b)
    ncopy.wait()
    for b in range(NBUF):
        fire_n(b, b)

    @pl.loop(0, NCHUNK // NBUF)
    def _round(t):
        c0 = t * NBUF
        for j in range(NBUF):
            c = c0 + j
            wait_gathers(j)
            compute(c, j)

            @pl.when(c + NBUF < NCHUNK)
            def _():
                fire(c + NBUF, j)

    pltpu.sync_copy(out_full, out_hbm.at[:, pl.ds(base, PER_W)])


_sc_cp = pltpu.CompilerParams()
if "needs_layout_passes" in pltpu.CompilerParams.__dataclass_fields__:
    _sc_cp = dataclasses.replace(_sc_cp, needs_layout_passes=False)

_sc_dots = pl.kernel(
    _sc_body,
    out_type=jax.ShapeDtypeStruct((ND, B), jnp.float32),
    mesh=plsc.VectorSubcoreMesh(core_axis_name="c", subcore_axis_name="s"),
    compiler_params=_sc_cp,
    scratch_types=(
        [
            pltpu.VMEM((PER_W,), jnp.int32),
            pltpu.VMEM((PER_W,), jnp.int32),
            pltpu.VMEM((NNEG, PER_W), jnp.int32),
            pltpu.VMEM((ND, PER_W), jnp.float32),
        ]
        + [
            pltpu.VMEM((CHUNK, DIM), jnp.float32),
            pltpu.VMEM((CHUNK, DIM), jnp.float32),
            pltpu.VMEM((NNEG, CHUNK, DIM), jnp.float32),
        ] * NBUF
        + [pltpu.SemaphoreType.DMA] * (NBUF + 1)
    ),
)


def _tc_finish_body(x_ref, o_ref):
    x = x_ref[...]
    x = jnp.clip(x, EPS, 1.0 - EPS)
    lp = -jnp.log(x[0:1, :])
    ln = -jnp.log(1.0 - x[1:ND, :])
    o_ref[0, 0] = (jnp.sum(lp) + jnp.sum(ln)) / B


_tc_finish = pl.pallas_call(
    _tc_finish_body,
    out_shape=jax.ShapeDtypeStruct((1, 1), jnp.float32),
    out_specs=pl.BlockSpec(memory_space=pltpu.SMEM),
)


@jax.jit
def kernel(pos_u, pos_v, neg_v, u_weight, v_weight):
    pos_u = pos_u.astype(jnp.int32)
    pos_v = pos_v.astype(jnp.int32)
    neg_t = jnp.transpose(neg_v.astype(jnp.int32))
    dots = _sc_dots(pos_u, pos_v, neg_t, u_weight, v_weight)
    return _tc_finish(dots)[0, 0]
